# Initial kernel scaffold; baseline (speedup 1.0000x reference)
import jax
import jax.numpy as jnp
from jax.experimental import pallas as pl


def kernel(node_scalars, edge_feats, d, W1, b1, W2, b2, ln_gamma, ln_beta, edge_index):
    raise NotImplementedError("write your pallas kernel here")


# speedup vs baseline: 2.2393x; 2.2393x over previous
import functools

import jax
import jax.numpy as jnp
from jax import lax
from jax.experimental import pallas as pl
from jax.experimental.pallas import tpu as pltpu
from jax.experimental.pallas import tpu_sc as plsc


def _tables_body(ns_ref, w1a_ref, w1b_ref, b1_ref, gs_ref, gd_ref):
    ns = ns_ref[...]
    gs_ref[...] = (
        jnp.dot(ns, w1a_ref[...], preferred_element_type=jnp.float32)
        + b1_ref[...]
    )
    gd_ref[...] = jnp.dot(ns, w1b_ref[...], preferred_element_type=jnp.float32)


def _make_tables(node_scalars, w1a, w1b, b1):
    n, s = node_scalars.shape
    f = w1a.shape[1]
    tn = 2000
    assert n % tn == 0
    grid = n // tn
    return pl.pallas_call(
        _tables_body,
        grid=(grid,),
        in_specs=[
            pl.BlockSpec((tn, s), lambda i: (i, 0)),
            pl.BlockSpec((s, f), lambda i: (0, 0)),
            pl.BlockSpec((s, f), lambda i: (0, 0)),
            pl.BlockSpec((1, f), lambda i: (0, 0)),
        ],
        out_specs=[
            pl.BlockSpec((tn, f), lambda i: (i, 0)),
            pl.BlockSpec((tn, f), lambda i: (i, 0)),
        ],
        out_shape=[
            jax.ShapeDtypeStruct((n, f), jnp.float32),
            jax.ShapeDtypeStruct((n, f), jnp.float32),
        ],
    )(node_scalars, w1a, w1b, b1.reshape(1, f))


_C = 128


def _gather_add(gs, gd, src, dst):
    e = src.shape[0]
    f = gs.shape[1]
    info = plsc.get_sparse_core_info()
    nw = info.num_cores * info.num_subcores
    n_chunks = e // _C
    assert e % _C == 0
    mesh = plsc.VectorSubcoreMesh(core_axis_name="c", subcore_axis_name="s")

    @functools.partial(
        pl.kernel,
        mesh=mesh,
        compiler_params=pltpu.CompilerParams(use_tc_tiling_on_sc=False),
        out_type=jax.ShapeDtypeStruct((e, f), jnp.float32),
        scratch_types=[
            pltpu.VMEM((_C,), jnp.int32),
            pltpu.VMEM((_C,), jnp.int32),
            pltpu.VMEM((_C, f), jnp.float32),
            pltpu.VMEM((_C, f), jnp.float32),
            pltpu.SemaphoreType.DMA,
            pltpu.SemaphoreType.DMA,
        ],
    )
    def k(gs_hbm, gd_hbm, src_hbm, dst_hbm, out_hbm,
          idx_s, idx_d, rows_s, rows_d, sem_s, sem_d):
        wid = lax.axis_index("s") * info.num_cores + lax.axis_index("c")
        my_chunks = n_chunks // nw + jnp.where(wid < n_chunks % nw, 1, 0)

        def chunk_body(i, carry):
            off = (wid + i * nw) * _C
            pltpu.sync_copy(src_hbm.at[pl.ds(off, _C)], idx_s)
            pltpu.sync_copy(dst_hbm.at[pl.ds(off, _C)], idx_d)
            cp_s = pltpu.async_copy(gs_hbm.at[idx_s], rows_s, sem_s)
            cp_d = pltpu.async_copy(gd_hbm.at[idx_d], rows_d, sem_d)
            cp_s.wait()
            cp_d.wait()

            def row_body(r, c2):
                for j in range(f // 16):
                    sl = pl.ds(j * 16, 16)
                    plsc.addupdate(rows_s.at[r, sl], rows_d[r, sl])
                return c2

            lax.fori_loop(0, _C, row_body, 0, unroll=2)
            pltpu.sync_copy(rows_s, out_hbm.at[pl.ds(off, _C)])
            return carry

        lax.fori_loop(0, my_chunks, chunk_body, 0)

    return k(gs, gd, src, dst)


def _mlp_body(ef_ref, d_ref, s_ref, w1c_ref, w1d_ref, w2_ref, b2_ref,
              g_ref, beta_ref, o_ref):
    ef = ef_ref[...]
    a = jnp.dot(ef, w1c_ref[...], preferred_element_type=jnp.float32)
    a = a + jnp.dot(d_ref[...], w1d_ref[...], preferred_element_type=jnp.float32)
    a = a + s_ref[...]
    h = a * jax.nn.sigmoid(a)
    h = jnp.dot(h, w2_ref[...], preferred_element_type=jnp.float32) + b2_ref[...]
    h = h * jax.nn.sigmoid(h)
    r = ef + h
    mu = jnp.mean(r, axis=-1, keepdims=True)
    c = r - mu
    var = jnp.mean(c * c, axis=-1, keepdims=True)
    o_ref[...] = c * lax.rsqrt(var + 1e-5) * g_ref[...] + beta_ref[...]


def _mlp(edge_feats, d, s_sum, w1c, w1d, w2, b2, gamma, beta):
    e, f = edge_feats.shape
    r = d.shape[1]
    te = 2000
    assert e % te == 0
    grid = e // te
    return pl.pallas_call(
        _mlp_body,
        grid=(grid,),
        in_specs=[
            pl.BlockSpec((te, f), lambda i: (i, 0)),
            pl.BlockSpec((te, r), lambda i: (i, 0)),
            pl.BlockSpec((te, f), lambda i: (i, 0)),
            pl.BlockSpec((f, f), lambda i: (0, 0)),
            pl.BlockSpec((r, f), lambda i: (0, 0)),
            pl.BlockSpec((f, f), lambda i: (0, 0)),
            pl.BlockSpec((1, f), lambda i: (0, 0)),
            pl.BlockSpec((1, f), lambda i: (0, 0)),
            pl.BlockSpec((1, f), lambda i: (0, 0)),
        ],
        out_specs=pl.BlockSpec((te, f), lambda i: (i, 0)),
        out_shape=jax.ShapeDtypeStruct((e, f), jnp.float32),
    )(edge_feats, d, s_sum, w1c, w1d, w2,
      b2.reshape(1, f), gamma.reshape(1, f), beta.reshape(1, f))


def kernel(node_scalars, edge_feats, d, W1, b1, W2, b2, ln_gamma, ln_beta,
           edge_index):
    s = node_scalars.shape[1]
    f = edge_feats.shape[1]
    w1a = W1[:s]
    w1b = W1[s:2 * s]
    w1c = W1[2 * s:2 * s + f]
    w1d = W1[2 * s + f:]
    gs, gd = _make_tables(node_scalars, w1a, w1b, b1)
    src = edge_index[0]
    dst = edge_index[1]
    s_sum = _gather_add(gs, gd, src, dst)
    return _mlp(edge_feats, d, s_sum, w1c, w1d, W2, b2, ln_gamma, ln_beta)


# speedup vs baseline: 2.7652x; 1.2349x over previous
import functools

import jax
import jax.numpy as jnp
from jax import lax
from jax.experimental import pallas as pl
from jax.experimental.pallas import tpu as pltpu
from jax.experimental.pallas import tpu_sc as plsc


def _tables_body(ns_ref, w1a_ref, w1b_ref, b1_ref, gs_ref, gd_ref):
    ns = ns_ref[...]
    gs_ref[...] = (
        jnp.dot(ns, w1a_ref[...], preferred_element_type=jnp.float32)
        + b1_ref[...]
    )
    gd_ref[...] = jnp.dot(ns, w1b_ref[...], preferred_element_type=jnp.float32)


def _make_tables(ns2, w1a2, w1b2, b1_2):
    n2 = ns2.shape[0]
    tn = 1000
    assert n2 % tn == 0
    grid = n2 // tn
    return pl.pallas_call(
        _tables_body,
        grid=(grid,),
        in_specs=[
            pl.BlockSpec((tn, 128), lambda i: (i, 0)),
            pl.BlockSpec((128, 128), lambda i: (0, 0)),
            pl.BlockSpec((128, 128), lambda i: (0, 0)),
            pl.BlockSpec((1, 128), lambda i: (0, 0)),
        ],
        out_specs=[
            pl.BlockSpec((tn, 128), lambda i: (i, 0)),
            pl.BlockSpec((tn, 128), lambda i: (i, 0)),
        ],
        out_shape=[
            jax.ShapeDtypeStruct((n2, 128), jnp.float32),
            jax.ShapeDtypeStruct((n2, 128), jnp.float32),
        ],
    )(ns2, w1a2, w1b2, b1_2)


_C = 128


def _gather_add(gs, gd, src, dst):
    e = src.shape[0]
    f = gs.shape[1]
    info = plsc.get_sparse_core_info()
    nw = info.num_cores * info.num_subcores
    n_chunks = e // _C
    assert e % _C == 0
    mesh = plsc.VectorSubcoreMesh(core_axis_name="c", subcore_axis_name="s")

    @functools.partial(
        pl.kernel,
        mesh=mesh,
        compiler_params=pltpu.CompilerParams(use_tc_tiling_on_sc=False),
        out_type=jax.ShapeDtypeStruct((e, f), jnp.float32),
        scratch_types=[
            pltpu.VMEM((_C,), jnp.int32),
            pltpu.VMEM((_C,), jnp.int32),
            pltpu.VMEM((_C, f), jnp.float32),
            pltpu.VMEM((_C, f), jnp.float32),
            pltpu.SemaphoreType.DMA,
            pltpu.SemaphoreType.DMA,
        ],
    )
    def k(gs_hbm, gd_hbm, src_hbm, dst_hbm, out_hbm,
          idx_s, idx_d, rows_s, rows_d, sem_s, sem_d):
        wid = lax.axis_index("s") * info.num_cores + lax.axis_index("c")
        my_chunks = n_chunks // nw + jnp.where(wid < n_chunks % nw, 1, 0)

        def chunk_body(i, carry):
            off = (wid + i * nw) * _C
            pltpu.sync_copy(src_hbm.at[pl.ds(off, _C)], idx_s)
            pltpu.sync_copy(dst_hbm.at[pl.ds(off, _C)], idx_d)
            cp_s = pltpu.async_copy(gs_hbm.at[idx_s], rows_s, sem_s)
            cp_d = pltpu.async_copy(gd_hbm.at[idx_d], rows_d, sem_d)
            cp_s.wait()
            cp_d.wait()

            def row_body(r, c2):
                for j in range(f // 16):
                    sl = pl.ds(j * 16, 16)
                    plsc.addupdate(rows_s.at[r, sl], rows_d[r, sl])
                return c2

            lax.fori_loop(0, _C, row_body, 0, unroll=2)
            pltpu.sync_copy(rows_s, out_hbm.at[pl.ds(off, _C)])
            return carry

        lax.fori_loop(0, my_chunks, chunk_body, 0)

    return k(gs, gd, src, dst)


def _mlp_body(ef_ref, d_ref, s_ref, w1c_ref, wd8_ref, w2_ref, b2_ref,
              g_ref, beta_ref, half_ref, o_ref):
    x = ef_ref[...]
    a = jnp.dot(x, w1c_ref[...], preferred_element_type=jnp.float32)
    yd = jnp.dot(d_ref[...], wd8_ref[...], preferred_element_type=jnp.float32)
    a = a + yd.reshape(x.shape) + s_ref[...]
    h = a * jax.nn.sigmoid(a)
    h = jnp.dot(h, w2_ref[...], preferred_element_type=jnp.float32) + b2_ref[...]
    h = h * jax.nn.sigmoid(h)
    r = x + h
    half = half_ref[...]
    mu = jnp.dot(r, half, preferred_element_type=jnp.float32)
    c = r - mu
    var = jnp.dot(c * c, half, preferred_element_type=jnp.float32)
    o_ref[...] = c * lax.rsqrt(var + 1e-5) * g_ref[...] + beta_ref[...]


def _mlp(ef2, d2, s2, w1c2, wd8, w2_2, b2_2, g2, beta2, half):
    e2 = ef2.shape[0]
    be2 = 4000
    be8 = be2 // 4
    assert e2 % be2 == 0
    grid = e2 // be2
    return pl.pallas_call(
        _mlp_body,
        grid=(grid,),
        in_specs=[
            pl.BlockSpec((be2, 128), lambda i: (i, 0)),
            pl.BlockSpec((be8, 128), lambda i: (i, 0)),
            pl.BlockSpec((be2, 128), lambda i: (i, 0)),
            pl.BlockSpec((128, 128), lambda i: (0, 0)),
            pl.BlockSpec((128, 512), lambda i: (0, 0)),
            pl.BlockSpec((128, 128), lambda i: (0, 0)),
            pl.BlockSpec((1, 128), lambda i: (0, 0)),
            pl.BlockSpec((1, 128), lambda i: (0, 0)),
            pl.BlockSpec((1, 128), lambda i: (0, 0)),
            pl.BlockSpec((128, 128), lambda i: (0, 0)),
        ],
        out_specs=pl.BlockSpec((be2, 128), lambda i: (i, 0)),
        out_shape=jax.ShapeDtypeStruct((e2, 128), jnp.float32),
    )(ef2, d2, s2, w1c2, wd8, w2_2, b2_2, g2, beta2, half)


def kernel(node_scalars, edge_feats, d, W1, b1, W2, b2, ln_gamma, ln_beta,
           edge_index):
    n, s = node_scalars.shape
    e, f = edge_feats.shape
    r = d.shape[1]
    w1a = W1[:s]
    w1b = W1[s:2 * s]
    w1c = W1[2 * s:2 * s + f]
    w1d = W1[2 * s + f:]

    eye2 = jnp.eye(2, dtype=jnp.float32)
    ns2 = node_scalars.reshape(n // 2, 2 * s)
    gs2, gd2 = _make_tables(
        ns2,
        jnp.kron(eye2, w1a),
        jnp.kron(eye2, w1b),
        jnp.concatenate([b1, b1]).reshape(1, 2 * f),
    )
    gs = gs2.reshape(n, f)
    gd = gd2.reshape(n, f)

    src = edge_index[0]
    dst = edge_index[1]
    s_sum = _gather_add(gs, gd, src, dst)

    npack = 128 // r
    half = jnp.kron(eye2, jnp.full((f, f), 1.0 / f, dtype=jnp.float32))
    out2 = _mlp(
        edge_feats.reshape(e // 2, 2 * f),
        d.reshape(e // npack, 128),
        s_sum.reshape(e // 2, 2 * f),
        jnp.kron(eye2, w1c),
        jnp.kron(jnp.eye(npack, dtype=jnp.float32), w1d),
        jnp.kron(eye2, W2),
        jnp.concatenate([b2, b2]).reshape(1, 2 * f),
        jnp.concatenate([ln_gamma, ln_gamma]).reshape(1, 2 * f),
        jnp.concatenate([ln_beta, ln_beta]).reshape(1, 2 * f),
        half,
    )
    return out2.reshape(e, f)


# speedup vs baseline: 3.2799x; 1.1861x over previous
import functools

import jax
import jax.numpy as jnp
from jax import lax
from jax.experimental import pallas as pl
from jax.experimental.pallas import tpu as pltpu
from jax.experimental.pallas import tpu_sc as plsc


_DNUM_T_LHS = (((0,), (0,)), ((), ()))


def _tables_body(nst_ref, w1a_ref, w1b_ref, b1_ref, gs_ref, gd_ref):
    nst = nst_ref[...]
    gs_ref[...] = (
        lax.dot_general(nst, w1a_ref[...], _DNUM_T_LHS,
                        preferred_element_type=jnp.float32)
        + b1_ref[...]
    )
    gd_ref[...] = lax.dot_general(nst, w1b_ref[...], _DNUM_T_LHS,
                                  preferred_element_type=jnp.float32)


def _make_tables(ns_t, w1a, w1b, b1_row):
    s, n = ns_t.shape
    f = w1a.shape[1]
    tn = 2048
    grid = (n + tn - 1) // tn
    return pl.pallas_call(
        _tables_body,
        grid=(grid,),
        in_specs=[
            pl.BlockSpec((s, tn), lambda i: (0, i)),
            pl.BlockSpec((s, f), lambda i: (0, 0)),
            pl.BlockSpec((s, f), lambda i: (0, 0)),
            pl.BlockSpec((1, f), lambda i: (0, 0)),
        ],
        out_specs=[
            pl.BlockSpec((tn, f), lambda i: (i, 0)),
            pl.BlockSpec((tn, f), lambda i: (i, 0)),
        ],
        out_shape=[
            jax.ShapeDtypeStruct((n, f), jnp.float32),
            jax.ShapeDtypeStruct((n, f), jnp.float32),
        ],
    )(ns_t, w1a, w1b, b1_row)


_C = 128


def _gather_add(gs, gd, src, dst):
    e = src.shape[0]
    f = gs.shape[1]
    info = plsc.get_sparse_core_info()
    nw = info.num_cores * info.num_subcores
    n_chunks = e // _C
    assert e % _C == 0
    mesh = plsc.VectorSubcoreMesh(core_axis_name="c", subcore_axis_name="s")

    @functools.partial(
        pl.kernel,
        mesh=mesh,
        compiler_params=pltpu.CompilerParams(use_tc_tiling_on_sc=False),
        out_type=jax.ShapeDtypeStruct((e, f), jnp.float32),
        scratch_types=[
            pltpu.VMEM((_C,), jnp.int32),
            pltpu.VMEM((_C,), jnp.int32),
            pltpu.VMEM((_C, f), jnp.float32),
            pltpu.VMEM((_C, f), jnp.float32),
            pltpu.SemaphoreType.DMA,
            pltpu.SemaphoreType.DMA,
        ],
    )
    def k(gs_hbm, gd_hbm, src_hbm, dst_hbm, out_hbm,
          idx_s, idx_d, rows_s, rows_d, sem_s, sem_d):
        wid = lax.axis_index("s") * info.num_cores + lax.axis_index("c")
        my_chunks = n_chunks // nw + jnp.where(wid < n_chunks % nw, 1, 0)

        def chunk_body(i, carry):
            off = (wid + i * nw) * _C
            pltpu.sync_copy(src_hbm.at[pl.ds(off, _C)], idx_s)
            pltpu.sync_copy(dst_hbm.at[pl.ds(off, _C)], idx_d)
            cp_s = pltpu.async_copy(gs_hbm.at[idx_s], rows_s, sem_s)
            cp_d = pltpu.async_copy(gd_hbm.at[idx_d], rows_d, sem_d)
            cp_s.wait()
            cp_d.wait()

            def row_body(r, c2):
                for j in range(f // 16):
                    sl = pl.ds(j * 16, 16)
                    plsc.addupdate(rows_s.at[r, sl], rows_d[r, sl])
                return c2

            lax.fori_loop(0, _C, row_body, 0, unroll=2)
            pltpu.sync_copy(rows_s, out_hbm.at[pl.ds(off, _C)])
            return carry

        lax.fori_loop(0, my_chunks, chunk_body, 0)

    return k(gs, gd, src, dst)


_DNUM_T_RHS = (((1,), (1,)), ((), ()))


def _mlp_body(xt_ref, dt_ref, s_ref, w1ct_ref, w1dt_ref, i_ref, w2t_ref,
              b2_ref, g_ref, beta_ref, o_ref):
    xt = xt_ref[...]
    st = lax.dot_general(i_ref[...], s_ref[...], _DNUM_T_RHS,
                         preferred_element_type=jnp.float32)
    a = jnp.dot(w1ct_ref[...], xt, preferred_element_type=jnp.float32)
    a = a + jnp.dot(w1dt_ref[...], dt_ref[...],
                    preferred_element_type=jnp.float32)
    a = a + st
    h = a * jax.nn.sigmoid(a)
    h = jnp.dot(w2t_ref[...], h, preferred_element_type=jnp.float32)
    h = h + b2_ref[...]
    h = h * jax.nn.sigmoid(h)
    r = xt + h
    mu = jnp.mean(r, axis=0, keepdims=True)
    c = r - mu
    var = jnp.mean(c * c, axis=0, keepdims=True)
    o_ref[...] = c * lax.rsqrt(var + 1e-5) * g_ref[...] + beta_ref[...]


def _mlp(ef_t, d_t, s_sum, w1c_t, w1d_t, i64, w2_t, b2_c, g_c, beta_c):
    f, e = ef_t.shape
    r = d_t.shape[0]
    te = 3200
    assert e % te == 0
    grid = e // te
    return pl.pallas_call(
        _mlp_body,
        grid=(grid,),
        in_specs=[
            pl.BlockSpec((f, te), lambda i: (0, i)),
            pl.BlockSpec((r, te), lambda i: (0, i)),
            pl.BlockSpec((te, f), lambda i: (i, 0)),
            pl.BlockSpec((f, f), lambda i: (0, 0)),
            pl.BlockSpec((f, r), lambda i: (0, 0)),
            pl.BlockSpec((f, f), lambda i: (0, 0)),
            pl.BlockSpec((f, f), lambda i: (0, 0)),
            pl.BlockSpec((f, 1), lambda i: (0, 0)),
            pl.BlockSpec((f, 1), lambda i: (0, 0)),
            pl.BlockSpec((f, 1), lambda i: (0, 0)),
        ],
        out_specs=pl.BlockSpec((f, te), lambda i: (0, i)),
        out_shape=jax.ShapeDtypeStruct((f, e), jnp.float32),
    )(ef_t, d_t, s_sum, w1c_t, w1d_t, i64, w2_t, b2_c, g_c, beta_c)


def kernel(node_scalars, edge_feats, d, W1, b1, W2, b2, ln_gamma, ln_beta,
           edge_index):
    n, s = node_scalars.shape
    e, f = edge_feats.shape
    w1a = W1[:s]
    w1b = W1[s:2 * s]
    w1c = W1[2 * s:2 * s + f]
    w1d = W1[2 * s + f:]

    gs, gd = _make_tables(node_scalars.T, w1a, w1b, b1.reshape(1, f))

    src = edge_index[0]
    dst = edge_index[1]
    s_sum = _gather_add(gs, gd, src, dst)

    out_t = _mlp(
        edge_feats.T,
        d.T,
        s_sum,
        w1c.T,
        w1d.T,
        jnp.eye(f, dtype=jnp.float32),
        W2.T,
        b2.reshape(f, 1),
        ln_gamma.reshape(f, 1),
        ln_beta.reshape(f, 1),
    )
    return out_t.T


# speedup vs baseline: 4.3662x; 1.3312x over previous
import functools

import jax
import jax.numpy as jnp
from jax import lax
from jax.experimental import pallas as pl
from jax.experimental.pallas import tpu as pltpu
from jax.experimental.pallas import tpu_sc as plsc


_DNUM_T_LHS = (((0,), (0,)), ((), ()))


def _tables_body(nst_ref, w1a_ref, w1b_ref, b1_ref, gs_ref, gd_ref):
    nst = nst_ref[...]
    gs_ref[...] = (
        lax.dot_general(nst, w1a_ref[...], _DNUM_T_LHS,
                        preferred_element_type=jnp.float32)
        + b1_ref[...]
    )
    gd_ref[...] = lax.dot_general(nst, w1b_ref[...], _DNUM_T_LHS,
                                  preferred_element_type=jnp.float32)


def _make_tables(ns_t, w1a, w1b, b1_row):
    s, n = ns_t.shape
    f = w1a.shape[1]
    tn = 2048
    grid = (n + tn - 1) // tn
    return pl.pallas_call(
        _tables_body,
        grid=(grid,),
        in_specs=[
            pl.BlockSpec((s, tn), lambda i: (0, i)),
            pl.BlockSpec((s, f), lambda i: (0, 0)),
            pl.BlockSpec((s, f), lambda i: (0, 0)),
            pl.BlockSpec((1, f), lambda i: (0, 0)),
        ],
        out_specs=[
            pl.BlockSpec((tn, f), lambda i: (i, 0)),
            pl.BlockSpec((tn, f), lambda i: (i, 0)),
        ],
        out_shape=[
            jax.ShapeDtypeStruct((n, f), jnp.float32),
            jax.ShapeDtypeStruct((n, f), jnp.float32),
        ],
    )(ns_t, w1a, w1b, b1_row)


_C = 128


def _gather_add(gs, gd, src, dst):
    e = src.shape[0]
    f = gs.shape[1]
    info = plsc.get_sparse_core_info()
    nw = info.num_cores * info.num_subcores
    n_chunks = e // _C
    assert e % _C == 0
    base_cnt = n_chunks // nw
    extra = n_chunks % nw
    maxc = base_cnt + (1 if extra else 0)
    mesh = plsc.VectorSubcoreMesh(core_axis_name="c", subcore_axis_name="s")

    @functools.partial(
        pl.kernel,
        mesh=mesh,
        compiler_params=pltpu.CompilerParams(use_tc_tiling_on_sc=False),
        out_type=jax.ShapeDtypeStruct((e, f), jnp.float32),
        scratch_types=[
            pltpu.VMEM((maxc * _C,), jnp.int32),
            pltpu.VMEM((maxc * _C,), jnp.int32),
            pltpu.VMEM((_C, f), jnp.float32),
            pltpu.VMEM((_C, f), jnp.float32),
            pltpu.VMEM((_C, f), jnp.float32),
            pltpu.VMEM((_C, f), jnp.float32),
            pltpu.SemaphoreType.DMA,
            pltpu.SemaphoreType.DMA,
        ],
    )
    def k(gs_hbm, gd_hbm, src_hbm, dst_hbm, out_hbm,
          ixs, ixd, rs_a, rd_a, rs_b, rd_b, sem_a, sem_b):
        wid = lax.axis_index("s") * info.num_cores + lax.axis_index("c")
        count = base_cnt + jnp.where(wid < extra, 1, 0)
        base_edge = (wid * base_cnt + jnp.minimum(wid, extra)) * _C
        nbase = base_cnt * _C
        pltpu.sync_copy(src_hbm.at[pl.ds(base_edge, nbase)],
                        ixs.at[pl.ds(0, nbase)])
        pltpu.sync_copy(dst_hbm.at[pl.ds(base_edge, nbase)],
                        ixd.at[pl.ds(0, nbase)])

        @pl.when(count > base_cnt)
        def _():
            pltpu.sync_copy(src_hbm.at[pl.ds(base_edge + nbase, _C)],
                            ixs.at[pl.ds(nbase, _C)])
            pltpu.sync_copy(dst_hbm.at[pl.ds(base_edge + nbase, _C)],
                            ixd.at[pl.ds(nbase, _C)])

        def fire(i, rs, rd, sem):
            o = i * _C
            pltpu.async_copy(gs_hbm.at[ixs.at[pl.ds(o, _C)]], rs, sem)
            pltpu.async_copy(gd_hbm.at[ixd.at[pl.ds(o, _C)]], rd, sem)

        def drain(rs, rd, sem):
            pltpu.make_async_copy(gs_hbm.at[pl.ds(0, _C)], rs, sem).wait()
            pltpu.make_async_copy(gs_hbm.at[pl.ds(0, _C)], rd, sem).wait()

        def process(i, rs, rd):
            def row_body(rr, c2):
                for j in range(f // 16):
                    sl = pl.ds(j * 16, 16)
                    plsc.addupdate(rs.at[rr, sl], rd[rr, sl])
                return c2

            lax.fori_loop(0, _C, row_body, 0, unroll=2)
            pltpu.sync_copy(rs, out_hbm.at[pl.ds(base_edge + i * _C, _C)])

        fire(0, rs_a, rd_a, sem_a)
        n_pairs = (count + 1) // 2

        def pair_body(j, carry):
            i0 = 2 * j
            i1 = i0 + 1

            @pl.when(i1 < count)
            def _():
                fire(i1, rs_b, rd_b, sem_b)

            drain(rs_a, rd_a, sem_a)
            process(i0, rs_a, rd_a)

            @pl.when(i1 + 1 < count)
            def _():
                fire(i1 + 1, rs_a, rd_a, sem_a)

            @pl.when(i1 < count)
            def _():
                drain(rs_b, rd_b, sem_b)
                process(i1, rs_b, rd_b)

            return carry

        lax.fori_loop(0, n_pairs, pair_body, 0)

    return k(gs, gd, src, dst)


_DNUM_T_RHS = (((1,), (1,)), ((), ()))


def _mlp_body(xt_ref, dt_ref, s_ref, w1ct_ref, w1dt_ref, i_ref, w2t_ref,
              b2_ref, g_ref, beta_ref, o_ref):
    xt = xt_ref[...]
    st = lax.dot_general(i_ref[...], s_ref[...], _DNUM_T_RHS,
                         preferred_element_type=jnp.float32)
    a = jnp.dot(w1ct_ref[...], xt, preferred_element_type=jnp.float32)
    a = a + jnp.dot(w1dt_ref[...], dt_ref[...],
                    preferred_element_type=jnp.float32)
    a = a + st
    h = a * jax.nn.sigmoid(a)
    h = jnp.dot(w2t_ref[...], h, preferred_element_type=jnp.float32)
    h = h + b2_ref[...]
    h = h * jax.nn.sigmoid(h)
    r = xt + h
    mu = jnp.mean(r, axis=0, keepdims=True)
    c = r - mu
    var = jnp.mean(c * c, axis=0, keepdims=True)
    o_ref[...] = c * lax.rsqrt(var + 1e-5) * g_ref[...] + beta_ref[...]


def _mlp(ef_t, d_t, s_sum, w1c_t, w1d_t, i64, w2_t, b2_c, g_c, beta_c):
    f, e = ef_t.shape
    r = d_t.shape[0]
    te = 3200
    assert e % te == 0
    grid = e // te
    return pl.pallas_call(
        _mlp_body,
        grid=(grid,),
        in_specs=[
            pl.BlockSpec((f, te), lambda i: (0, i)),
            pl.BlockSpec((r, te), lambda i: (0, i)),
            pl.BlockSpec((te, f), lambda i: (i, 0)),
            pl.BlockSpec((f, f), lambda i: (0, 0)),
            pl.BlockSpec((f, r), lambda i: (0, 0)),
            pl.BlockSpec((f, f), lambda i: (0, 0)),
            pl.BlockSpec((f, f), lambda i: (0, 0)),
            pl.BlockSpec((f, 1), lambda i: (0, 0)),
            pl.BlockSpec((f, 1), lambda i: (0, 0)),
            pl.BlockSpec((f, 1), lambda i: (0, 0)),
        ],
        out_specs=pl.BlockSpec((f, te), lambda i: (0, i)),
        out_shape=jax.ShapeDtypeStruct((f, e), jnp.float32),
    )(ef_t, d_t, s_sum, w1c_t, w1d_t, i64, w2_t, b2_c, g_c, beta_c)


def kernel(node_scalars, edge_feats, d, W1, b1, W2, b2, ln_gamma, ln_beta,
           edge_index):
    n, s = node_scalars.shape
    e, f = edge_feats.shape
    w1a = W1[:s]
    w1b = W1[s:2 * s]
    w1c = W1[2 * s:2 * s + f]
    w1d = W1[2 * s + f:]

    gs, gd = _make_tables(node_scalars.T, w1a, w1b, b1.reshape(1, f))

    src = edge_index[0]
    dst = edge_index[1]
    s_sum = _gather_add(gs, gd, src, dst)

    out_t = _mlp(
        edge_feats.T,
        d.T,
        s_sum,
        w1c.T,
        w1d.T,
        jnp.eye(f, dtype=jnp.float32),
        W2.T,
        b2.reshape(f, 1),
        ln_gamma.reshape(f, 1),
        ln_beta.reshape(f, 1),
    )
    return out_t.T
